# baseline (speedup 1.0000x reference)
import jax
import jax.numpy as jnp
from jax import lax
from jax.experimental import pallas as pl
from jax.experimental.pallas import tpu as pltpu
from jax.experimental.pallas import tpu_sc as plsc

_W = 128
_NW = 32


def kernel(ids, table):
    B, S = ids.shape
    V, D = table.shape
    assert (B, S, D) == (16384, 200, 32)

    ids = ids.astype(jnp.int32)
    i5 = ids.T.reshape(S // 8, 8, B // _W, _W).transpose(0, 2, 1, 3)

    n_win = S * (B // _W)
    per_w = n_win // _NW

    mesh = plsc.VectorSubcoreMesh(core_axis_name="c", subcore_axis_name="s")
    cp = pltpu.CompilerParams(
        use_tc_tiling_on_sc=False, needs_layout_passes=False
    )

    @jax.jit
    def run(table_arr, idx_arr):
        @pl.kernel(
            out_type=jax.ShapeDtypeStruct((S, D // 8, B // _W, 8, _W),
                                          table_arr.dtype),
            mesh=mesh,
            compiler_params=cp,
            scratch_types=[
                pltpu.VMEM((2, _W), jnp.int32),
                pltpu.VMEM((2, _W, D), table_arr.dtype),
                pltpu.VMEM((2, D // 8, 8, _W), table_arr.dtype),
                pltpu.SemaphoreType.DMA,
                pltpu.SemaphoreType.DMA,
                pltpu.SemaphoreType.DMA,
                pltpu.SemaphoreType.DMA,
                pltpu.SemaphoreType.DMA,
                pltpu.SemaphoreType.DMA,
            ],
        )
        def k(table_hbm, i_hbm, o_hbm, idx_v, r_v, t_v,
              is0, is1, gs0, gs1, os0, os1):
            isem = (is0, is1)
            gsem = (gs0, gs1)
            osem = (os0, os1)
            wid = lax.axis_index("s") * 2 + lax.axis_index("c")
            base = wid * per_w

            lane = lax.iota(jnp.int32, 16)
            rows = [lane + 16 * c for c in range(_W // 16)]
            cols = [jnp.full((16,), e, jnp.int32) for e in range(D)]

            def coords(t):
                win_id = base + t
                s = win_id // (B // _W)
                bt = win_id % (B // _W)
                return s // 8, bt, s % 8, s

            def idx_src(t):
                st, bt, ss, _ = coords(t)
                return i_hbm.at[st, bt, ss]

            def out_dst(t):
                _, bt, _, s = coords(t)
                return o_hbm.at[s, :, bt, :, :]

            def start_idx(p, t):
                pltpu.async_copy(idx_src(t), idx_v.at[p], isem[p])

            def wait_idx(p, t):
                pltpu.make_async_copy(idx_src(t), idx_v.at[p], isem[p]).wait()

            def start_gather(p):
                pltpu.async_copy(table_hbm.at[idx_v.at[p]], r_v.at[p], gsem[p])

            def wait_gather(p):
                pltpu.make_async_copy(
                    table_hbm.at[idx_v.at[p]], r_v.at[p], gsem[p]
                ).wait()

            def start_out(p, t):
                pltpu.async_copy(t_v.at[p], out_dst(t), osem[p])

            def wait_out(p, t):
                pltpu.make_async_copy(t_v.at[p], out_dst(t), osem[p]).wait()

            def transpose(p):
                for c in range(_W // 16):
                    vals = [plsc.load_gather(r_v.at[p], [rows[c], cols[e]])
                            for e in range(D)]
                    for e in range(D):
                        t_v[p, e // 8, e % 8, pl.ds(16 * c, 16)] = vals[e]

            start_idx(0, 0)
            start_idx(1, 1)
            wait_idx(0, 0)
            start_gather(0)

            @pl.loop(0, per_w, step=2)
            def _(t):
                wait_idx(1, t + 1)
                start_gather(1)
                wait_gather(0)

                @pl.when(t >= 2)
                def _():
                    wait_out(0, t - 2)

                transpose(0)
                start_out(0, t)

                @pl.when(t + 2 < per_w)
                def _():
                    start_idx(0, t + 2)

                wait_gather(1)

                @pl.when(t >= 2)
                def _():
                    wait_out(1, t - 1)

                transpose(1)
                start_out(1, t + 1)

                @pl.when(t + 2 < per_w)
                def _():
                    wait_idx(0, t + 2)
                    start_gather(0)

                @pl.when(t + 3 < per_w)
                def _():
                    start_idx(1, t + 3)

            wait_out(0, per_w - 2)
            wait_out(1, per_w - 1)

        return k(table_arr, idx_arr)

    f = run(table, i5)
    return f.transpose(2, 4, 0, 1, 3).reshape(B, S, D)


# speedup vs baseline: 1.0232x; 1.0232x over previous
import jax
import jax.numpy as jnp
from jax import lax
from jax.experimental import pallas as pl
from jax.experimental.pallas import tpu as pltpu
from jax.experimental.pallas import tpu_sc as plsc

_W = 128
_SS = 4


def kernel(ids, table):
    B, S = ids.shape
    V, D = table.shape
    assert (B, S, D) == (16384, 200, 32)

    ids = ids.astype(jnp.int32)
    i5 = ids.T.reshape(S // 8, 8, B // _W, _W).transpose(0, 2, 1, 3)

    mesh = plsc.VectorSubcoreMesh(core_axis_name="c", subcore_axis_name="s")
    cp = pltpu.CompilerParams(
        use_tc_tiling_on_sc=False, needs_layout_passes=False
    )

    @jax.jit
    def run(table_arr, idx_arr):
        @pl.kernel(
            out_type=jax.ShapeDtypeStruct((S, D // 8, B // _W, 8, _W),
                                          table_arr.dtype),
            mesh=mesh,
            compiler_params=cp,
            scratch_types=[pltpu.VMEM((_SS, _W, D), table_arr.dtype)]
            + [pltpu.SemaphoreType.DMA] * _SS,
        )
        def k(table_hbm, i_hbm, o_hbm, r_v, *sems):
            lane = lax.iota(jnp.int32, 16)
            rows = [lane + 16 * c for c in range(_W // 16)]
            cols = [jnp.full((16,), e, jnp.int32) for e in range(D)]

            def body(i_ref, o_ref):
                for j in range(_SS):
                    pltpu.async_copy(
                        table_hbm.at[i_ref.at[0, 0, j]], r_v.at[j], sems[j]
                    )
                for j in range(_SS):
                    pltpu.make_async_copy(
                        table_hbm.at[i_ref.at[0, 0, j]], r_v.at[j], sems[j]
                    ).wait()
                    for c in range(_W // 16):
                        vals = [plsc.load_gather(r_v.at[j], [rows[c], cols[e]])
                                for e in range(D)]
                        for e in range(D):
                            o_ref[j, e // 8, 0, e % 8, pl.ds(16 * c, 16)] = (
                                vals[e])

            pltpu.emit_pipeline(
                body,
                grid=(S // 8 * (8 // _SS), B // _W),
                in_specs=[pl.BlockSpec(
                    (1, 1, _SS, _W),
                    index_map=lambda sq, bt: (sq // (8 // _SS), bt,
                                              sq % (8 // _SS), 0))],
                out_specs=[pl.BlockSpec(
                    (_SS, D // 8, 1, 8, _W),
                    index_map=lambda sq, bt: (sq, 0, bt, 0, 0))],
                core_axis_name=("c", "s"),
                dimension_semantics=(pltpu.PARALLEL, pltpu.PARALLEL),
            )(i_hbm, o_hbm)

        return k(table_arr, idx_arr)

    f = run(table, i5)
    return f.transpose(2, 4, 0, 1, 3).reshape(B, S, D)
